# Initial kernel scaffold; baseline (speedup 1.0000x reference)
import jax
import jax.numpy as jnp
from jax.experimental import pallas as pl


def kernel(x, attn_w, gate_w, W1, b1, W2, b2, wq, bq, wk, bk, wv, bv, wo, bo, f1w, f1b, f2w, f2b):
    raise NotImplementedError("write your pallas kernel here")


# speedup vs baseline: 1.7672x; 1.7672x over previous
import functools
import math

import jax
import jax.numpy as jnp
from jax.experimental import pallas as pl

D = 1024
E = 8
H = 1024
DOUT = 1024
FFN = 4096
NHEADS = 8
TOPK = 2
HD = DOUT // NHEADS


def _pool_gate_body(x_ref, attn_w_ref, gate_w_ref, xf_ref, comb_ref):
    aw = attn_w_ref[...]
    dn = (((1,), (0,)), ((), ()))
    O = x_ref.shape[1]
    xo = [x_ref[:, o, :] for o in range(O)]
    ls = [jax.lax.dot_general(xi, aw, dn, preferred_element_type=jnp.float32)
          for xi in xo]
    m = ls[0]
    for l in ls[1:]:
        m = jnp.maximum(m, l)
    es = [jnp.exp(l - m) for l in ls]
    ssum = es[0]
    for e_ in es[1:]:
        ssum = ssum + e_
    xa = xo[0] * (es[0] / ssum)
    for o in range(1, O):
        xa = xa + xo[o] * (es[o] / ssum)
    xf_ref[...] = xa

    g = jax.lax.dot_general(xa, gate_w_ref[...],
                            (((1,), (0,)), ((), ())),
                            preferred_element_type=jnp.float32)
    tb = g.shape[0]
    iota = jax.lax.broadcasted_iota(jnp.int32, (tb, E), 1)
    m1 = jnp.max(g, axis=1, keepdims=True)
    i1 = jnp.min(jnp.where(g == m1, iota, E), axis=1, keepdims=True)
    mask1 = iota == i1
    neg = jnp.full_like(g, -jnp.inf)
    g2 = jnp.where(mask1, neg, g)
    m2 = jnp.max(g2, axis=1, keepdims=True)
    i2 = jnp.min(jnp.where(g2 == m2, iota, E), axis=1, keepdims=True)
    mask2 = iota == i2
    d = jnp.exp(m2 - m1)
    w1 = 1.0 / (1.0 + d)
    w2 = d * w1
    comb_ref[...] = (mask1.astype(jnp.float32) * w1
                     + mask2.astype(jnp.float32) * w2)


def _pool_gate(x4, attn_w, gate_w):
    T = x4.shape[0]
    O = x4.shape[1]
    TB = 256
    grid = (T // TB,)
    return pl.pallas_call(
        _pool_gate_body,
        grid=grid,
        in_specs=[
            pl.BlockSpec((TB, O, D), lambda i: (i, 0, 0)),
            pl.BlockSpec((D, 1), lambda i: (0, 0)),
            pl.BlockSpec((D, E), lambda i: (0, 0)),
        ],
        out_specs=[
            pl.BlockSpec((TB, D), lambda i: (i, 0)),
            pl.BlockSpec((TB, E), lambda i: (i, 0)),
        ],
        out_shape=[
            jax.ShapeDtypeStruct((T, D), jnp.float32),
            jax.ShapeDtypeStruct((T, E), jnp.float32),
        ],
    )(x4, attn_w, gate_w)


def _moe_body(xf_ref, comb_ref, W1_ref, b1_ref, W2_ref, b2_ref, out_ref):
    e = pl.program_id(0)
    xf = xf_ref[...]
    b1 = b1_ref[pl.ds(e, 1), :]
    b2 = b2_ref[pl.ds(e, 1), :]
    h = jax.lax.dot_general(xf, W1_ref[0], (((1,), (0,)), ((), ())),
                            preferred_element_type=jnp.float32) + b1
    h = jnp.maximum(h, 0.0)
    y = jax.lax.dot_general(h, W2_ref[0], (((1,), (0,)), ((), ())),
                            preferred_element_type=jnp.float32) + b2
    comb = comb_ref[...]
    sel = (jax.lax.broadcasted_iota(jnp.int32, comb.shape, 1) == e)
    c = jnp.sum(jnp.where(sel, comb, 0.0), axis=1, keepdims=True)
    contrib = c * y

    @pl.when(e == 0)
    def _():
        out_ref[...] = contrib

    @pl.when(e != 0)
    def _():
        out_ref[...] = out_ref[...] + contrib


def _moe(xf, comb, W1, b1, W2, b2):
    T = xf.shape[0]
    return pl.pallas_call(
        _moe_body,
        grid=(E,),
        in_specs=[
            pl.BlockSpec((T, D), lambda e: (0, 0)),
            pl.BlockSpec((T, E), lambda e: (0, 0)),
            pl.BlockSpec((1, D, H), lambda e: (e, 0, 0)),
            pl.BlockSpec((E, H), lambda e: (0, 0)),
            pl.BlockSpec((1, H, DOUT), lambda e: (e, 0, 0)),
            pl.BlockSpec((E, DOUT), lambda e: (0, 0)),
        ],
        out_specs=pl.BlockSpec((T, DOUT), lambda e: (0, 0)),
        out_shape=jax.ShapeDtypeStruct((T, DOUT), jnp.float32),
    )(xf, comb, W1, b1, W2, b2)


def _attn_body(ef_ref, wq_ref, bq_ref, wk_ref, bk_ref, wv_ref, bv_ref,
               wo_ref, bo_ref, out_ref):
    ef = ef_ref[0]
    dn = (((1,), (0,)), ((), ()))
    q = jax.lax.dot_general(ef, wq_ref[...], dn,
                            preferred_element_type=jnp.float32) + bq_ref[...]
    k = jax.lax.dot_general(ef, wk_ref[...], dn,
                            preferred_element_type=jnp.float32) + bk_ref[...]
    v = jax.lax.dot_general(ef, wv_ref[...], dn,
                            preferred_element_type=jnp.float32) + bv_ref[...]
    scale = 1.0 / math.sqrt(HD)
    outs = []
    for hh in range(NHEADS):
        sl = slice(hh * HD, (hh + 1) * HD)
        qh = q[:, sl]
        kh = k[:, sl]
        vh = v[:, sl]
        s = jax.lax.dot_general(qh, kh, (((1,), (1,)), ((), ())),
                                preferred_element_type=jnp.float32) * scale
        m = jnp.max(s, axis=1, keepdims=True)
        p = jnp.exp(s - m)
        p = p / jnp.sum(p, axis=1, keepdims=True)
        outs.append(jax.lax.dot_general(p, vh, dn,
                                        preferred_element_type=jnp.float32))
    o = jnp.concatenate(outs, axis=1)
    out_ref[0] = jax.lax.dot_general(o, wo_ref[...], dn,
                                     preferred_element_type=jnp.float32) + bo_ref[...]


def _attn(ef, wq, bq, wk, bk, wv, bv, wo, bo):
    B, N, _ = ef.shape
    wspec = pl.BlockSpec((DOUT, DOUT), lambda b: (0, 0))
    bspec = pl.BlockSpec((1, DOUT), lambda b: (0, 0))
    return pl.pallas_call(
        _attn_body,
        grid=(B,),
        in_specs=[
            pl.BlockSpec((1, N, DOUT), lambda b: (b, 0, 0)),
            wspec, bspec, wspec, bspec, wspec, bspec, wspec, bspec,
        ],
        out_specs=pl.BlockSpec((1, N, DOUT), lambda b: (b, 0, 0)),
        out_shape=jax.ShapeDtypeStruct((B, N, DOUT), jnp.float32),
    )(ef, wq, bq.reshape(1, DOUT), wk, bk.reshape(1, DOUT),
      wv, bv.reshape(1, DOUT), wo, bo.reshape(1, DOUT))


def _ffn_body(x_ref, f1w_ref, f1b_ref, f2w_ref, f2b_ref, out_ref):
    dn = (((1,), (0,)), ((), ()))
    h = jax.lax.dot_general(x_ref[...], f1w_ref[...], dn,
                            preferred_element_type=jnp.float32) + f1b_ref[...]
    h = jnp.maximum(h, 0.0)
    out_ref[...] = jax.lax.dot_general(h, f2w_ref[...], dn,
                                       preferred_element_type=jnp.float32) + f2b_ref[...]


def _ffn(x2, f1w, f1b, f2w, f2b):
    T = x2.shape[0]
    TB = 256
    return pl.pallas_call(
        _ffn_body,
        grid=(T // TB,),
        in_specs=[
            pl.BlockSpec((TB, DOUT), lambda i: (i, 0)),
            pl.BlockSpec((DOUT, FFN), lambda i: (0, 0)),
            pl.BlockSpec((1, FFN), lambda i: (0, 0)),
            pl.BlockSpec((FFN, DOUT), lambda i: (0, 0)),
            pl.BlockSpec((1, DOUT), lambda i: (0, 0)),
        ],
        out_specs=pl.BlockSpec((TB, DOUT), lambda i: (i, 0)),
        out_shape=jax.ShapeDtypeStruct((T, DOUT), jnp.float32),
    )(x2, f1w, f1b.reshape(1, FFN), f2w, f2b.reshape(1, DOUT))


@jax.jit
def kernel(x, attn_w, gate_w, W1, b1, W2, b2, wq, bq, wk, bk, wv, bv,
           wo, bo, f1w, f1b, f2w, f2b):
    B, N, O, d = x.shape
    T = B * N
    x4 = x.reshape(T, O, d)
    xf, comb = _pool_gate(x4, attn_w, gate_w)
    moe = _moe(xf, comb, W1, b1, W2, b2)
    ef = moe.reshape(B, N, DOUT)
    rel = _attn(ef, wq, bq, wk, bk, wv, bv, wo, bo)
    out = _ffn(rel.reshape(T, DOUT), f1w, f1b, f2w, f2b)
    return out.reshape(B, N, DOUT)
